# Initial kernel scaffold; baseline (speedup 1.0000x reference)
import jax
import jax.numpy as jnp
from jax.experimental import pallas as pl


def kernel(w_big, w_mid, w_small, bias, output_embedding, mem_bank, mem_padding_mask, scores, save_period):
    raise NotImplementedError("write your pallas kernel here")


# speedup vs baseline: 1.1330x; 1.1330x over previous
import functools

import numpy as np

import jax
import jax.numpy as jnp
from jax.experimental import pallas as pl
from jax.experimental.pallas import tpu as pltpu

_D = 128
_HID = 512
_H = 8
_L = 4
_HD = _D // _H
_LD = _L * _D

_BN = 256

_NEG = -1e9


def _round_up(x, m):
  return ((x + m - 1) // m) * m


def _np_hb():
  hb = np.zeros((_LD, _L * _H), np.float32)
  for l in range(_L):
    for d in range(_D):
      hb[l * _D + d, l * _H + d // _HD] = 1.0
  return hb


def _np_ex():
  ex = np.zeros((_L * _H, _LD), np.float32)
  for l in range(_L):
    for d in range(_D):
      ex[l * _H + d // _HD, l * _D + d] = 1.0
  return ex


_HB = _np_hb()
_EX = _np_ex()


def _mb_kernel(x_ref, mem_ref, meta_ref, wa_ref, w2_ref, hb_ref, ex_ref,
               b_ref, bank_ref, xo_ref, mo_ref, *, eps, save_thresh,
               save_period_const):
  f32 = jnp.float32
  x = x_ref[...]
  mem = mem_ref[...]
  meta = meta_ref[...]

  wkv = wa_ref[:, 0:256]
  wq = wa_ref[:, 256:384]
  wo = wa_ref[:, 384:512]
  w1 = wa_ref[:, 512:1024]
  ws = wa_ref[:, 1024:1152]

  b1 = b_ref[0:1, :]
  bk = b_ref[1:2, 0:128]
  bv = b_ref[1:2, 128:256]
  bq = b_ref[1:2, 256:384]
  bo = b_ref[1:2, 384:512]
  b2 = b_ref[2:3, 0:128]
  g1 = b_ref[2:3, 128:256]
  be1 = b_ref[2:3, 256:384]
  g2 = b_ref[2:3, 384:512]
  be2 = b_ref[3:4, 0:128]
  bs = b_ref[3:4, 128:256]

  q = jnp.dot(x, wq, preferred_element_type=f32) + bq

  ks, vs = [], []
  for l in range(_L):
    kv = jnp.dot(mem[:, l * _D:(l + 1) * _D], wkv, preferred_element_type=f32)
    ks.append(kv[:, 0:_D] + bk)
    vs.append(kv[:, _D:2 * _D] + bv)

  e = jnp.concatenate([q * k for k in ks], axis=1)
  s32 = jnp.dot(e, hb_ref[...], preferred_element_type=f32)
  madd = jnp.concatenate(
      [jnp.broadcast_to(meta[:, l:l + 1], (meta.shape[0], _H))
       for l in range(_L)], axis=1) * _NEG
  s32 = s32 + madd

  m8 = jnp.maximum(jnp.maximum(s32[:, 0:8], s32[:, 8:16]),
                   jnp.maximum(s32[:, 16:24], s32[:, 24:32]))
  p32 = jnp.exp(s32 - jnp.concatenate([m8] * _L, axis=1))
  d8 = (p32[:, 0:8] + p32[:, 8:16]) + (p32[:, 16:24] + p32[:, 24:32])
  inv8 = pl.reciprocal(d8, approx=False)
  pn = p32 * jnp.concatenate([inv8] * _L, axis=1)

  pe = jnp.dot(pn, ex_ref[...], preferred_element_type=f32)
  ctx = (pe[:, 0:128] * vs[0] + pe[:, 128:256] * vs[1]
         + pe[:, 256:384] * vs[2] + pe[:, 384:512] * vs[3])
  emb = jnp.dot(ctx, wo, preferred_element_type=f32) + bo

  def layer_norm(v, g, b):
    mu = jnp.mean(v, axis=-1, keepdims=True)
    c = v - mu
    var = jnp.mean(c * c, axis=-1, keepdims=True)
    return c * jax.lax.rsqrt(var + eps) * g + b

  e1 = layer_norm(x + emb, g1, be1)
  hh = jnp.maximum(jnp.dot(e1, w1, preferred_element_type=f32) + b1, 0.0)
  ff = jnp.dot(hh, w2_ref[...], preferred_element_type=f32) + b2
  e2 = layer_norm(e1 + ff, g2, be2)

  valid = meta[:, 3:4] == 0.0
  new_x = jnp.where(valid, e2, x)

  score = meta[:, 4:5]
  sp = meta[:, 5:6]
  saved = jnp.logical_and(sp == 0.0, score > save_thresh)
  new_sp = jnp.where(sp > 0.0, sp - 1.0, sp)
  new_sp = jnp.where(saved, jnp.float32(save_period_const), new_sp)

  se = jnp.dot(new_x, ws, preferred_element_type=f32) + bs
  shifted = jnp.concatenate([mem[:, _D:_LD], se], axis=1)
  bank_ref[...] = jnp.where(saved, shifted, mem)
  xo_ref[...] = new_x

  mask_sh = jnp.concatenate(
      [meta[:, 1:4], jnp.zeros_like(meta[:, 0:1])], axis=1)
  new_mask = jnp.where(saved, mask_sh, meta[:, 0:4])
  mo_ref[...] = jnp.concatenate(
      [new_mask, new_sp, jnp.zeros_like(meta[:, 0:3])], axis=1)


def kernel(w_big, w_mid, w_small, bias, output_embedding, mem_bank,
           mem_padding_mask, scores, save_period):
  f32 = jnp.float32
  N = output_embedding.shape[0]

  wk_t = w_big[0:128, 0:128]
  wq_ts = w_big[512:640, 512:640]
  wv_t = w_big[0:128, 1024:1152]
  bk = w_big[650:651, 0:128]
  bq_s = w_big[650:651, 512:640]
  bv = w_big[650:651, 1024:1152]
  wo_t = w_mid[0:128, 2048:2176]
  w1_t = w_small[0, 0:128, 0:512]
  w2_t = w_small[1, 0:512, 0:128]
  ws_t = w_small[2, 0:128, 384:512]
  bo = bias[0:1, 0:128]
  b1 = bias[1:2, 0:512]
  b2 = bias[2:3, 0:128]
  g1 = bias[3:4, 0:128]
  be1 = bias[4:5, 0:128]
  g2 = bias[5:6, 0:128]
  be2 = bias[6:7, 0:128]
  bs = bias[7:8, 384:512]

  wa = jnp.concatenate([wk_t, wv_t, wq_ts, wo_t, w1_t, ws_t], axis=1)
  zero128 = jnp.zeros((1, 128), f32)
  b_pack = jnp.concatenate([
      b1,
      jnp.concatenate([bk, bv, bq_s, bo], axis=1),
      jnp.concatenate([b2, g1, be1, g2], axis=1),
      jnp.concatenate([be2, bs, zero128, zero128], axis=1),
      jnp.zeros((4, 512), f32),
  ], axis=0)

  x = output_embedding.astype(f32)
  mem = mem_bank.reshape(N, _LD).astype(f32)
  meta = jnp.concatenate([
      mem_padding_mask.reshape(N, _L).astype(f32),
      scores.reshape(N, 1).astype(f32),
      save_period.reshape(N, 1).astype(f32),
      jnp.zeros((N, 2), f32),
  ], axis=1)

  bn = _BN if N >= _BN else _round_up(max(N, 8), 8)
  n_pad = _round_up(N, bn)
  if n_pad > N:
    pad = ((0, n_pad - N), (0, 0))
    x = jnp.pad(x, pad)
    mem = jnp.pad(mem, pad)
    meta = jnp.pad(meta, pad)
  grid = (n_pad // bn,)

  kfn = functools.partial(_mb_kernel, eps=1e-5, save_thresh=0.4,
                          save_period_const=3)

  bank, xo, mo = pl.pallas_call(
      kfn,
      grid=grid,
      in_specs=[
          pl.BlockSpec((bn, _D), lambda i: (i, 0)),
          pl.BlockSpec((bn, _LD), lambda i: (i, 0)),
          pl.BlockSpec((bn, 8), lambda i: (i, 0)),
          pl.BlockSpec((128, 1152), lambda i: (0, 0)),
          pl.BlockSpec((512, 128), lambda i: (0, 0)),
          pl.BlockSpec((_LD, _L * _H), lambda i: (0, 0)),
          pl.BlockSpec((_L * _H, _LD), lambda i: (0, 0)),
          pl.BlockSpec((8, 512), lambda i: (0, 0)),
      ],
      out_specs=[
          pl.BlockSpec((bn, _LD), lambda i: (i, 0)),
          pl.BlockSpec((bn, _D), lambda i: (i, 0)),
          pl.BlockSpec((bn, 8), lambda i: (i, 0)),
      ],
      out_shape=[
          jax.ShapeDtypeStruct((n_pad, _LD), f32),
          jax.ShapeDtypeStruct((n_pad, _D), f32),
          jax.ShapeDtypeStruct((n_pad, 8), f32),
      ],
      compiler_params=pltpu.CompilerParams(
          dimension_semantics=("parallel",)),
  )(x, mem, meta, wa, w2_t, jnp.asarray(_HB), jnp.asarray(_EX), b_pack)

  new_bank = bank[:N].reshape(N, _L, _D)
  new_x = xo[:N]
  new_mask = mo[:N, 0:_L] > 0.5
  new_sp = mo[:N, _L].astype(jnp.int32)
  return new_x, new_bank, new_mask, new_sp


# speedup vs baseline: 1.2191x; 1.0760x over previous
import functools

import numpy as np

import jax
import jax.numpy as jnp
from jax import lax
from jax.experimental import pallas as pl
from jax.experimental.pallas import tpu as pltpu

_D = 128
_HID = 512
_H = 8
_L = 4
_HD = _D // _H
_LD = _L * _D

_BN = 256

_NEG = -1e9


def _round_up(x, m):
  return ((x + m - 1) // m) * m


def _np_hb():
  hb = np.zeros((_LD, _L * _H), np.float32)
  for l in range(_L):
    for d in range(_D):
      hb[l * _D + d, l * _H + d // _HD] = 1.0
  return hb


def _np_ex():
  ex = np.zeros((_L * _H, _LD), np.float32)
  for l in range(_L):
    for d in range(_D):
      ex[l * _H + d // _HD, l * _D + d] = 1.0
  return ex


_HB = _np_hb()
_EX = _np_ex()


def _mb_kernel(x_ref, mem_ref, mask_ref, sc_ref, sp_ref,
               wk_ref, wq_ref, wv_ref, brow_ref, wo_ref,
               w1_ref, w2_ref, ws_ref, bias_ref, hb_ref, ex_ref,
               bank_ref, xo_ref, mo_ref, spo_ref,
               *, eps, save_thresh, save_period_const):
  f32 = jnp.float32
  x = x_ref[...]
  mem = mem_ref[...]
  mask = mask_ref[...]
  bn = x.shape[0]

  bk = brow_ref[2:3, 0:128]
  bq = brow_ref[2:3, 512:640]
  bv = brow_ref[2:3, 1024:1152]
  w1 = w1_ref[0]
  w2 = w2_ref[0]
  ws = ws_ref[0]
  bo = bias_ref[0:1, 0:128]
  b1 = bias_ref[1:2, 0:512]
  b2 = bias_ref[2:3, 0:128]
  g1 = bias_ref[3:4, 0:128]
  be1 = bias_ref[4:5, 0:128]
  g2 = bias_ref[5:6, 0:128]
  be2 = bias_ref[6:7, 0:128]
  bs = bias_ref[7:8, 384:512]

  G = bn // 128
  sc_l = sc_ref[0]
  sp_l = sp_ref[0].astype(f32)
  r_i = lax.broadcasted_iota(jnp.int32, (bn, 128), 0)
  l_i = lax.broadcasted_iota(jnp.int32, (bn, 128), 1)
  p_sel = (l_i == r_i % 128).astype(f32)
  sc_rows = jnp.concatenate(
      [jnp.broadcast_to(sc_l[g:g + 1, :], (128, 128)) for g in range(G)],
      axis=0)
  sp_rows = jnp.concatenate(
      [jnp.broadcast_to(sp_l[g:g + 1, :], (128, 128)) for g in range(G)],
      axis=0)
  score = jnp.sum(sc_rows * p_sel, axis=1, keepdims=True)
  sp = jnp.sum(sp_rows * p_sel, axis=1, keepdims=True)

  q = jnp.dot(x, wq_ref[...], preferred_element_type=f32) + bq

  wkv = jnp.concatenate([wk_ref[...], wv_ref[...]], axis=1)
  ks, vs = [], []
  for l in range(_L):
    kv = jnp.dot(mem[:, l * _D:(l + 1) * _D], wkv, preferred_element_type=f32)
    ks.append(kv[:, 0:_D] + bk)
    vs.append(kv[:, _D:2 * _D] + bv)

  e = jnp.concatenate([q * k for k in ks], axis=1)
  s32 = jnp.dot(e, hb_ref[...], preferred_element_type=f32)
  madd = jnp.concatenate(
      [jnp.broadcast_to(mask[:, l:l + 1], (bn, _H)) for l in range(_L)],
      axis=1) * _NEG
  s32 = s32 + madd

  m8 = jnp.maximum(jnp.maximum(s32[:, 0:8], s32[:, 8:16]),
                   jnp.maximum(s32[:, 16:24], s32[:, 24:32]))
  p32 = jnp.exp(s32 - jnp.concatenate([m8] * _L, axis=1))
  d8 = (p32[:, 0:8] + p32[:, 8:16]) + (p32[:, 16:24] + p32[:, 24:32])
  inv8 = pl.reciprocal(d8, approx=False)
  pn = p32 * jnp.concatenate([inv8] * _L, axis=1)

  pe = jnp.dot(pn, ex_ref[...], preferred_element_type=f32)
  ctx = (pe[:, 0:128] * vs[0] + pe[:, 128:256] * vs[1]
         + pe[:, 256:384] * vs[2] + pe[:, 384:512] * vs[3])
  emb = jnp.dot(ctx, wo_ref[...], preferred_element_type=f32) + bo

  def layer_norm(v, g, b):
    mu = jnp.mean(v, axis=-1, keepdims=True)
    cc = v - mu
    var = jnp.mean(cc * cc, axis=-1, keepdims=True)
    return cc * jax.lax.rsqrt(var + eps) * g + b

  e1 = layer_norm(x + emb, g1, be1)
  hh = jnp.maximum(jnp.dot(e1, w1, preferred_element_type=f32) + b1, 0.0)
  ff = jnp.dot(hh, w2, preferred_element_type=f32) + b2
  e2 = layer_norm(e1 + ff, g2, be2)

  valid = mask[:, 3:4] == 0.0
  new_x = jnp.where(valid, e2, x)

  saved = jnp.logical_and(sp == 0.0, score > save_thresh)
  new_sp = jnp.where(sp > 0.0, sp - 1.0, sp)
  new_sp = jnp.where(saved, jnp.float32(save_period_const), new_sp)

  se = jnp.dot(new_x, ws, preferred_element_type=f32) + bs
  shifted = jnp.concatenate([mem[:, _D:_LD], se], axis=1)
  bank_ref[...] = jnp.where(saved, shifted, mem)
  xo_ref[...] = new_x

  mask_sh = jnp.concatenate(
      [mask[:, 1:4], jnp.zeros_like(mask[:, 0:1])], axis=1)
  mo_ref[...] = jnp.where(saved, mask_sh, mask)

  col_p = new_sp * p_sel
  spo_ref[0] = jnp.concatenate(
      [jnp.sum(col_p[g * 128:(g + 1) * 128, :], axis=0, keepdims=True)
       for g in range(G)], axis=0).astype(jnp.int32)


def kernel(w_big, w_mid, w_small, bias, output_embedding, mem_bank,
           mem_padding_mask, scores, save_period):
  f32 = jnp.float32
  N = output_embedding.shape[0]

  x = output_embedding.astype(f32)
  mem = mem_bank.reshape(N, _LD).astype(f32)
  mask = mem_padding_mask.astype(f32)

  bn = _BN
  n_pad = _round_up(N, bn)
  sc = scores.astype(f32)
  sp = save_period
  if n_pad > N:
    pad = ((0, n_pad - N), (0, 0))
    x = jnp.pad(x, pad)
    mem = jnp.pad(mem, pad)
    mask = jnp.pad(mask, pad)
    sc = jnp.pad(sc, (0, n_pad - N))
    sp = jnp.pad(sp, (0, n_pad - N))
  grid = (n_pad // bn,)
  gb = bn // 128
  sc2 = sc.reshape(n_pad // bn, gb, 128)
  sp2 = sp.reshape(n_pad // bn, gb, 128)

  kfn = functools.partial(_mb_kernel, eps=1e-5, save_thresh=0.4,
                          save_period_const=3)

  bank, xo, mo, spo = pl.pallas_call(
      kfn,
      grid=grid,
      in_specs=[
          pl.BlockSpec((bn, _D), lambda i: (i, 0)),
          pl.BlockSpec((bn, _LD), lambda i: (i, 0)),
          pl.BlockSpec((bn, _L), lambda i: (i, 0)),
          pl.BlockSpec((1, gb, 128), lambda i: (i, 0, 0)),
          pl.BlockSpec((1, gb, 128), lambda i: (i, 0, 0)),
          pl.BlockSpec((128, 128), lambda i: (0, 0)),
          pl.BlockSpec((128, 128), lambda i: (4, 4)),
          pl.BlockSpec((128, 128), lambda i: (0, 8)),
          pl.BlockSpec((8, 2048), lambda i: (81, 0)),
          pl.BlockSpec((128, 128), lambda i: (0, 16)),
          pl.BlockSpec((1, 128, 512), lambda i: (0, 0, 0)),
          pl.BlockSpec((1, 512, 128), lambda i: (1, 0, 0)),
          pl.BlockSpec((1, 128, 128), lambda i: (2, 0, 3)),
          pl.BlockSpec((8, 512), lambda i: (0, 0)),
          pl.BlockSpec((_LD, _L * _H), lambda i: (0, 0)),
          pl.BlockSpec((_L * _H, _LD), lambda i: (0, 0)),
      ],
      out_specs=[
          pl.BlockSpec((bn, _LD), lambda i: (i, 0)),
          pl.BlockSpec((bn, _D), lambda i: (i, 0)),
          pl.BlockSpec((bn, _L), lambda i: (i, 0)),
          pl.BlockSpec((1, gb, 128), lambda i: (i, 0, 0)),
      ],
      out_shape=[
          jax.ShapeDtypeStruct((n_pad, _LD), f32),
          jax.ShapeDtypeStruct((n_pad, _D), f32),
          jax.ShapeDtypeStruct((n_pad, _L), f32),
          jax.ShapeDtypeStruct((n_pad // bn, gb, 128), jnp.int32),
      ],
      compiler_params=pltpu.CompilerParams(
          dimension_semantics=("parallel",)),
  )(x, mem, mask, sc2, sp2, w_big, w_big, w_big, w_big, w_mid,
    w_small, w_small, w_small, bias, jnp.asarray(_HB), jnp.asarray(_EX))

  new_bank = bank[:N].reshape(N, _L, _D)
  new_x = xo[:N]
  new_mask = mo[:N] > 0.5
  new_sp = spo.reshape(n_pad)[:N]
  return new_x, new_bank, new_mask, new_sp


# speedup vs baseline: 1.9321x; 1.5849x over previous
import functools

import numpy as np

import jax
import jax.numpy as jnp
from jax import lax
from jax.experimental import pallas as pl
from jax.experimental.pallas import tpu as pltpu

_D = 128
_HID = 512
_H = 8
_L = 4
_HD = _D // _H
_LD = _L * _D

_BN = 256

_NEG = -1e9


def _round_up(x, m):
  return ((x + m - 1) // m) * m


def _np_hb():
  hb = np.zeros((_LD, _L * _H), np.float32)
  for l in range(_L):
    for d in range(_D):
      hb[l * _D + d, l * _H + d // _HD] = 1.0
  return hb


def _np_ex():
  ex = np.zeros((_L * _H, _LD), np.float32)
  for l in range(_L):
    for d in range(_D):
      ex[l * _H + d // _HD, l * _D + d] = 1.0
  return ex


_HB = _np_hb()
_EX = _np_ex()


def _mb_kernel(x_ref, mem_ref, mask_ref, sc_ref, sp_ref,
               wk_ref, wq_ref, wv_ref, brow_ref, wo_ref,
               w1_ref, w2_ref, ws_ref, bias_ref, hb_ref, ex_ref,
               bank_ref, xo_ref, mo_ref, spo_ref,
               *, eps, save_thresh, save_period_const):
  f32 = jnp.float32
  x = x_ref[...]
  mem3 = mem_ref[...]
  mems = [mem3[:, l, :] for l in range(_L)]
  mask = mask_ref[...]
  bn = x.shape[0]

  bk = brow_ref[2:3, 0:128]
  bq = brow_ref[2:3, 512:640]
  bv = brow_ref[2:3, 1024:1152]
  w1 = w1_ref[0]
  w2 = w2_ref[0]
  ws = ws_ref[0]
  bo = bias_ref[0:1, 0:128]
  b1 = bias_ref[1:2, 0:512]
  b2 = bias_ref[2:3, 0:128]
  g1 = bias_ref[3:4, 0:128]
  be1 = bias_ref[4:5, 0:128]
  g2 = bias_ref[5:6, 0:128]
  be2 = bias_ref[6:7, 0:128]
  bs = bias_ref[7:8, 384:512]

  G = bn // 128
  sc_l = sc_ref[0]
  sp_l = sp_ref[0].astype(f32)
  r_i = lax.broadcasted_iota(jnp.int32, (bn, 128), 0)
  l_i = lax.broadcasted_iota(jnp.int32, (bn, 128), 1)
  p_sel = (l_i == r_i % 128).astype(f32)
  sc_rows = jnp.concatenate(
      [jnp.broadcast_to(sc_l[g:g + 1, :], (128, 128)) for g in range(G)],
      axis=0)
  sp_rows = jnp.concatenate(
      [jnp.broadcast_to(sp_l[g:g + 1, :], (128, 128)) for g in range(G)],
      axis=0)
  score = jnp.sum(sc_rows * p_sel, axis=1, keepdims=True)
  sp = jnp.sum(sp_rows * p_sel, axis=1, keepdims=True)

  q = jnp.dot(x, wq_ref[...], preferred_element_type=f32) + bq

  wkv = jnp.concatenate([wk_ref[...], wv_ref[...]], axis=1)
  ks, vs = [], []
  for l in range(_L):
    kv = jnp.dot(mems[l], wkv, preferred_element_type=f32)
    ks.append(kv[:, 0:_D] + bk)
    vs.append(kv[:, _D:2 * _D] + bv)

  e = jnp.concatenate([q * k for k in ks], axis=1)
  s32 = jnp.dot(e, hb_ref[...], preferred_element_type=f32)
  madd = jnp.concatenate(
      [jnp.broadcast_to(mask[:, l:l + 1], (bn, _H)) for l in range(_L)],
      axis=1) * _NEG
  s32 = s32 + madd

  m8 = jnp.maximum(jnp.maximum(s32[:, 0:8], s32[:, 8:16]),
                   jnp.maximum(s32[:, 16:24], s32[:, 24:32]))
  p32 = jnp.exp(s32 - jnp.concatenate([m8] * _L, axis=1))
  d8 = (p32[:, 0:8] + p32[:, 8:16]) + (p32[:, 16:24] + p32[:, 24:32])
  inv8 = pl.reciprocal(d8, approx=False)
  pn = p32 * jnp.concatenate([inv8] * _L, axis=1)

  pe = jnp.dot(pn, ex_ref[...], preferred_element_type=f32)
  ctx = (pe[:, 0:128] * vs[0] + pe[:, 128:256] * vs[1]
         + pe[:, 256:384] * vs[2] + pe[:, 384:512] * vs[3])
  emb = jnp.dot(ctx, wo_ref[...], preferred_element_type=f32) + bo

  def layer_norm(v, g, b):
    mu = jnp.mean(v, axis=-1, keepdims=True)
    cc = v - mu
    var = jnp.mean(cc * cc, axis=-1, keepdims=True)
    return cc * jax.lax.rsqrt(var + eps) * g + b

  e1 = layer_norm(x + emb, g1, be1)
  hh = jnp.maximum(jnp.dot(e1, w1, preferred_element_type=f32) + b1, 0.0)
  ff = jnp.dot(hh, w2, preferred_element_type=f32) + b2
  e2 = layer_norm(e1 + ff, g2, be2)

  valid = mask[:, 3:4] == 0.0
  new_x = jnp.where(valid, e2, x)

  saved = jnp.logical_and(sp == 0.0, score > save_thresh)
  new_sp = jnp.where(sp > 0.0, sp - 1.0, sp)
  new_sp = jnp.where(saved, jnp.float32(save_period_const), new_sp)

  se = jnp.dot(new_x, ws, preferred_element_type=f32) + bs
  nexts = [mems[1], mems[2], mems[3], se]
  for l in range(_L):
    bank_ref[:, l, :] = jnp.where(saved, nexts[l], mems[l])
  xo_ref[...] = new_x

  mask_sh = jnp.concatenate(
      [mask[:, 1:4], jnp.zeros_like(mask[:, 0:1])], axis=1)
  mo_ref[...] = jnp.where(saved, mask_sh, mask)

  col_p = new_sp * p_sel
  spo_ref[0] = jnp.concatenate(
      [jnp.sum(col_p[g * 128:(g + 1) * 128, :], axis=0, keepdims=True)
       for g in range(G)], axis=0).astype(jnp.int32)


def kernel(w_big, w_mid, w_small, bias, output_embedding, mem_bank,
           mem_padding_mask, scores, save_period):
  f32 = jnp.float32
  N = output_embedding.shape[0]

  x = output_embedding.astype(f32)
  mem = mem_bank.astype(f32)
  mask = mem_padding_mask.astype(f32)

  bn = _BN
  n_pad = _round_up(N, bn)
  sc = scores.astype(f32)
  sp = save_period
  if n_pad > N:
    pad = ((0, n_pad - N), (0, 0))
    x = jnp.pad(x, pad)
    mem = jnp.pad(mem, ((0, n_pad - N), (0, 0), (0, 0)))
    mask = jnp.pad(mask, pad)
    sc = jnp.pad(sc, (0, n_pad - N))
    sp = jnp.pad(sp, (0, n_pad - N))
  grid = (n_pad // bn,)
  gb = bn // 128
  sc2 = sc.reshape(n_pad // bn, gb, 128)
  sp2 = sp.reshape(n_pad // bn, gb, 128)

  kfn = functools.partial(_mb_kernel, eps=1e-5, save_thresh=0.4,
                          save_period_const=3)

  bank, xo, mo, spo = pl.pallas_call(
      kfn,
      grid=grid,
      in_specs=[
          pl.BlockSpec((bn, _D), lambda i: (i, 0)),
          pl.BlockSpec((bn, _L, _D), lambda i: (i, 0, 0)),
          pl.BlockSpec((bn, _L), lambda i: (i, 0)),
          pl.BlockSpec((1, gb, 128), lambda i: (i, 0, 0)),
          pl.BlockSpec((1, gb, 128), lambda i: (i, 0, 0)),
          pl.BlockSpec((128, 128), lambda i: (0, 0)),
          pl.BlockSpec((128, 128), lambda i: (4, 4)),
          pl.BlockSpec((128, 128), lambda i: (0, 8)),
          pl.BlockSpec((8, 2048), lambda i: (81, 0)),
          pl.BlockSpec((128, 128), lambda i: (0, 16)),
          pl.BlockSpec((1, 128, 512), lambda i: (0, 0, 0)),
          pl.BlockSpec((1, 512, 128), lambda i: (1, 0, 0)),
          pl.BlockSpec((1, 128, 128), lambda i: (2, 0, 3)),
          pl.BlockSpec((8, 512), lambda i: (0, 0)),
          pl.BlockSpec((_LD, _L * _H), lambda i: (0, 0)),
          pl.BlockSpec((_L * _H, _LD), lambda i: (0, 0)),
      ],
      out_specs=[
          pl.BlockSpec((bn, _L, _D), lambda i: (i, 0, 0)),
          pl.BlockSpec((bn, _D), lambda i: (i, 0)),
          pl.BlockSpec((bn, _L), lambda i: (i, 0)),
          pl.BlockSpec((1, gb, 128), lambda i: (i, 0, 0)),
      ],
      out_shape=[
          jax.ShapeDtypeStruct((n_pad, _L, _D), f32),
          jax.ShapeDtypeStruct((n_pad, _D), f32),
          jax.ShapeDtypeStruct((n_pad, _L), f32),
          jax.ShapeDtypeStruct((n_pad // bn, gb, 128), jnp.int32),
      ],
      compiler_params=pltpu.CompilerParams(
          dimension_semantics=("parallel",)),
  )(x, mem, mask, sc2, sp2, w_big, w_big, w_big, w_big, w_mid,
    w_small, w_small, w_small, bias, jnp.asarray(_HB), jnp.asarray(_EX))

  new_bank = bank[:N]
  new_x = xo[:N]
  new_mask = mo[:N] > 0.5
  new_sp = spo.reshape(n_pad)[:N]
  return new_x, new_bank, new_mask, new_sp
